# baseline (speedup 1.0000x reference)
import functools

import jax
import jax.numpy as jnp
from jax import lax
from jax.experimental import pallas as pl
from jax.experimental.pallas import tpu as pltpu
from jax.experimental.pallas import tpu_sc as plsc

_NUM_CLASSES = 100000
_MULT = 3
_EMBED = 64
_ROW = 128

_BL = 4096 * 200
_B = _BL * _MULT
_NC = 2
_NS = 16
_NW = _NC * _NS
_PER_W = _B // _NW
_G = 768
_NGRP = _PER_W // _G
_NT = _G // 128


def _sc_gather(feat_flat, table_pad):
  mesh = plsc.VectorSubcoreMesh(core_axis_name="c", subcore_axis_name="s")

  @functools.partial(
      pl.kernel,
      mesh=mesh,
      out_type=jax.ShapeDtypeStruct((_BL * 8, _ROW), jnp.float32),
      scratch_types=[
          pltpu.VMEM((_G,), jnp.int32),
          pltpu.VMEM((_NT, 128), jnp.int32),
          pltpu.VMEM((_NT, 128), jnp.int32),
          pltpu.VMEM((_G, _ROW), jnp.float32),
          pltpu.SemaphoreType.DMA,
          pltpu.SemaphoreType.DMA,
      ],
  )
  def k(feat_hbm, table_hbm, out_hbm, feat_v, idx_v, oidx_v, rows_v,
        gsem, ssem):
    wid = lax.axis_index("s") * _NC + lax.axis_index("c")
    lane = lax.iota(jnp.int32, 16)

    def body(g, carry):
      base = wid * _PER_W + g * _G
      pltpu.sync_copy(feat_hbm.at[pl.ds(base, _G)], feat_v)
      for s in range(_G // 16):
        p = base + (s * 16) + lane
        r3 = (lane + (s * 16) % 3) % 3
        idx_v[s // 8, pl.ds((s % 8) * 16, 16)] = (
            feat_v[pl.ds(s * 16, 16)] + r3 * _NUM_CLASSES)
        q3 = (p - r3) * jnp.int32(-1431655765)
        oidx_v[s // 8, pl.ds((s % 8) * 16, 16)] = q3 * 8 + r3
      gathers = [
          pltpu.async_copy(
              table_hbm.at[idx_v.at[t]],
              rows_v.at[pl.ds(t * 128, 128)],
              gsem,
          )
          for t in range(_NT)
      ]
      for c in gathers:
        c.wait()
      scatters = [
          pltpu.async_copy(
              rows_v.at[pl.ds(t * 128, 128)],
              out_hbm.at[oidx_v.at[t]],
              ssem,
          )
          for t in range(_NT)
      ]
      for c in scatters:
        c.wait()
      return carry

    lax.fori_loop(0, _NGRP, body, 0)

  return k(feat_flat, table_pad)


def kernel(feat, table):
  feat_flat = feat.reshape(-1)
  table_pad = jnp.pad(table, ((0, 0), (0, _ROW - _EMBED)))
  out2 = _sc_gather(feat_flat, table_pad)
  out4 = out2.reshape(4096, 200, 8, _ROW)[:, :, :_MULT, :_EMBED]
  return out4


# speedup vs baseline: 4.9695x; 4.9695x over previous
import functools

import jax
import jax.numpy as jnp
from jax import lax
from jax.experimental import pallas as pl
from jax.experimental.pallas import tpu as pltpu
from jax.experimental.pallas import tpu_sc as plsc

_NUM_CLASSES = 100000
_MULT = 3
_EMBED = 64

_NB = 4096
_NL = 200
_NSLAB = _NL * _MULT
_B = _NB * _NSLAB
_NC = 2
_NSC = 16
_NW = _NC * _NSC
_G = 512
_NT = _G // 128
_HB = _NB // 2
_K = 5
_CSLAB = _NSLAB // _K
_CUNITS = _CSLAB * (_NB // _G)
_PER_W = _CUNITS // _NW
_NSET = 3
_NIT = _PER_W // _NSET


def _unit_params(u, k):
  h = u >> 3
  l = (h * 43691) >> 17
  m = h - 3 * l
  bc = u & 7
  foff = (m * _NL + l) * _NB + bc * _G
  rb = (h - k * _CSLAB) * _HB + (bc & 3) * _G
  cb = (bc >> 2) * _EMBED
  bias = m * _NUM_CLASSES
  return foff, rb, cb, bias


def _sc_gather_chunk(feat_t, table, k):
  mesh = plsc.VectorSubcoreMesh(core_axis_name="c", subcore_axis_name="s")

  scratch = []
  for _ in range(_NSET):
    scratch += [
        pltpu.VMEM((_G,), jnp.int32),
        pltpu.VMEM((_NT, 128), jnp.int32),
        pltpu.VMEM((_G, _EMBED), jnp.float32),
        pltpu.SemaphoreType.DMA,
        pltpu.SemaphoreType.DMA,
        pltpu.SemaphoreType.DMA,
    ]

  @functools.partial(
      pl.kernel,
      mesh=mesh,
      out_type=jax.ShapeDtypeStruct((_CSLAB * _HB, 128), jnp.float32),
      scratch_types=scratch,
      compiler_params=pltpu.CompilerParams(use_tc_tiling_on_sc=False),
  )
  def body(feat_hbm, table_hbm, out_hbm, *bufs):
    sets = [bufs[i * 6:(i + 1) * 6] for i in range(_NSET)]
    wid = lax.axis_index("s") * _NC + lax.axis_index("c")
    u0 = k * _CUNITS + wid * _PER_W

    def prefetch_feat(u, feat_v, fsem):
      foff, _, _, _ = _unit_params(u, k)
      pltpu.async_copy(feat_hbm.at[pl.ds(foff, _G)], feat_v, fsem)

    def fire_gathers(idx_v, rows_v, gsem):
      for t in range(_NT):
        pltpu.async_copy(
            table_hbm.at[idx_v.at[t]],
            rows_v.at[pl.ds(t * 128, 128)],
            gsem,
        )

    def drain_gathers(rows_v, gsem):
      pltpu.make_async_copy(table_hbm.at[pl.ds(0, _G)], rows_v, gsem).wait()

    for s in range(_NSET):
      prefetch_feat(u0 + s, sets[s][0], sets[s][3])

    def it(i, carry):
      for s in range(_NSET):
        feat_v, idx_v, rows_v, fsem, gsem, wsem = sets[s]
        u = u0 + i * _NSET + s
        foff, rb, cb, bias = _unit_params(u, k)
        pltpu.make_async_copy(feat_hbm.at[pl.ds(foff, _G)], feat_v,
                              fsem).wait()
        for q in range(_G // 16):
          idx_v[q // 8, pl.ds((q % 8) * 16, 16)] = (
              feat_v[pl.ds(q * 16, 16)] + bias)

        @pl.when(i < _NIT - 1)
        def _():
          prefetch_feat(u + _NSET, feat_v, fsem)

        @pl.when(i > 0)
        def _():
          _, prb, pcb, _ = _unit_params(u - _NSET, k)
          pltpu.make_async_copy(
              rows_v,
              out_hbm.at[pl.ds(prb, _G), pl.ds(pcb, _EMBED)],
              wsem).wait()

        fire_gathers(idx_v, rows_v, gsem)

        ps = (s + _NSET - 1) % _NSET
        pfeat_v, pidx_v, prows_v, pfsem, pgsem, pwsem = sets[ps]

        @pl.when((i > 0) | (s > 0))
        def _():
          pu = u - 1
          _, prb, pcb, _ = _unit_params(pu, k)
          drain_gathers(prows_v, pgsem)
          pltpu.async_copy(
              prows_v,
              out_hbm.at[pl.ds(prb, _G), pl.ds(pcb, _EMBED)],
              pwsem)
      return carry

    lax.fori_loop(0, _NIT, it, 0)

    lu = u0 + _PER_W - 1
    lfeat_v, lidx_v, lrows_v, lfsem, lgsem, lwsem = sets[_NSET - 1]
    lfoff, lrb, lcb, _ = _unit_params(lu, k)
    drain_gathers(lrows_v, lgsem)
    pltpu.async_copy(
        lrows_v, out_hbm.at[pl.ds(lrb, _G), pl.ds(lcb, _EMBED)], lwsem)
    for s in range(_NSET):
      feat_v, idx_v, rows_v, fsem, gsem, wsem = sets[s]
      u = u0 + (_NIT - 1) * _NSET + s
      _, rb, cb, _ = _unit_params(u, k)
      pltpu.make_async_copy(
          rows_v, out_hbm.at[pl.ds(rb, _G), pl.ds(cb, _EMBED)], wsem).wait()

  return body(feat_t, table)


def _tc_transpose_first(x_ref, o_ref):
  y = x_ref[0].T
  o_ref[0, :, :_HB] = y[:_EMBED]
  o_ref[0, :, _HB:] = y[_EMBED:]


def _tc_transpose_next(acc_ref, x_ref, o_ref):
  del acc_ref
  _tc_transpose_first(x_ref, o_ref)


def _tc_transpose(g3, k, acc):
  out_spec = pl.BlockSpec(
      (1, _EMBED, _NB), lambda i, _k=k: (_k * _CSLAB + i, 0, 0))
  x_spec = pl.BlockSpec((1, _HB, 128), lambda i: (i, 0, 0))
  out_shape = jax.ShapeDtypeStruct((_NSLAB, _EMBED, _NB), jnp.float32)
  if acc is None:
    return pl.pallas_call(
        _tc_transpose_first,
        grid=(_CSLAB,),
        in_specs=[x_spec],
        out_specs=out_spec,
        out_shape=out_shape,
    )(g3)
  return pl.pallas_call(
      _tc_transpose_next,
      grid=(_CSLAB,),
      in_specs=[pl.BlockSpec(memory_space=pl.ANY), x_spec],
      out_specs=out_spec,
      out_shape=out_shape,
      input_output_aliases={0: 0},
  )(acc, g3)


def kernel(feat, table):
  feat_t = feat.transpose(2, 1, 0).reshape(-1)
  acc = None
  for k in range(_K):
    g = _sc_gather_chunk(feat_t, table, k)
    g3 = g.reshape(_CSLAB, _HB, 128)
    acc = _tc_transpose(g3, k, acc)
  return acc.reshape(_NL, _MULT, _EMBED, _NB).transpose(3, 0, 1, 2)


# speedup vs baseline: 5.6131x; 1.1295x over previous
import functools

import jax
import jax.numpy as jnp
from jax import lax
from jax.experimental import pallas as pl
from jax.experimental.pallas import tpu as pltpu
from jax.experimental.pallas import tpu_sc as plsc

_NUM_CLASSES = 100000
_MULT = 3
_EMBED = 64

_NB = 4096
_NL = 200
_NSLAB = _NL * _MULT
_B = _NB * _NSLAB
_NC = 2
_NSC = 16
_NW = _NC * _NSC
_G = 512
_NT = _G // 128
_HB = _NB // 2
_K = 5
_CSLAB = _NSLAB // _K
_CUNITS = _CSLAB * (_NB // _G)
_PER_W = _CUNITS // _NW
_NSET = 3
_NIT = _PER_W // _NSET


def _unit_params(u, k):
  h = u >> 3
  l = (h * 43691) >> 17
  m = h - 3 * l
  bc = u & 7
  foff = (m * _NL + l) * _NB + bc * _G
  rb = (h - k * _CSLAB) * _HB + (bc & 3) * _G
  cb = (bc >> 2) * _EMBED
  bias = m * _NUM_CLASSES
  return foff, rb, cb, bias


def _sc_gather_chunk(feat_t, table, k):
  mesh = plsc.VectorSubcoreMesh(core_axis_name="c", subcore_axis_name="s")

  scratch = []
  for _ in range(_NSET):
    scratch += [
        pltpu.VMEM((_G,), jnp.int32),
        pltpu.VMEM((_NT, 128), jnp.int32),
        pltpu.VMEM((_G, _EMBED), jnp.float32),
        pltpu.SemaphoreType.DMA,
        pltpu.SemaphoreType.DMA,
        pltpu.SemaphoreType.DMA,
    ]

  @functools.partial(
      pl.kernel,
      mesh=mesh,
      out_type=jax.ShapeDtypeStruct((_CSLAB * _HB, 128), jnp.float32),
      scratch_types=scratch,
      compiler_params=pltpu.CompilerParams(use_tc_tiling_on_sc=False),
  )
  def body(feat_hbm, table_hbm, out_hbm, *bufs):
    sets = [bufs[i * 6:(i + 1) * 6] for i in range(_NSET)]
    wid = lax.axis_index("s") * _NC + lax.axis_index("c")
    u0 = k * _CUNITS + wid * _PER_W

    def prefetch_feat(u, feat_v, fsem):
      foff, _, _, _ = _unit_params(u, k)
      pltpu.async_copy(feat_hbm.at[pl.ds(foff, _G)], feat_v, fsem)

    def fire_gathers(idx_v, rows_v, gsem):
      for t in range(_NT):
        pltpu.async_copy(
            table_hbm.at[idx_v.at[t]],
            rows_v.at[pl.ds(t * 128, 128)],
            gsem,
        )

    def drain_gathers(rows_v, gsem):
      pltpu.make_async_copy(table_hbm.at[pl.ds(0, _G)], rows_v, gsem).wait()

    for s in range(_NSET):
      prefetch_feat(u0 + s, sets[s][0], sets[s][3])

    def it(i, carry):
      for s in range(_NSET):
        feat_v, idx_v, rows_v, fsem, gsem, wsem = sets[s]
        u = u0 + i * _NSET + s
        foff, rb, cb, bias = _unit_params(u, k)
        pltpu.make_async_copy(feat_hbm.at[pl.ds(foff, _G)], feat_v,
                              fsem).wait()
        for q in range(_G // 16):
          idx_v[q // 8, pl.ds((q % 8) * 16, 16)] = (
              feat_v[pl.ds(q * 16, 16)] + bias)

        @pl.when(i < _NIT - 1)
        def _():
          prefetch_feat(u + _NSET, feat_v, fsem)

        @pl.when(i > 0)
        def _():
          _, prb, pcb, _ = _unit_params(u - _NSET, k)
          pltpu.make_async_copy(
              rows_v,
              out_hbm.at[pl.ds(prb, _G), pl.ds(pcb, _EMBED)],
              wsem).wait()

        fire_gathers(idx_v, rows_v, gsem)

        ps = (s + _NSET - 1) % _NSET
        pfeat_v, pidx_v, prows_v, pfsem, pgsem, pwsem = sets[ps]

        @pl.when((i > 0) | (s > 0))
        def _():
          pu = u - 1
          _, prb, pcb, _ = _unit_params(pu, k)
          drain_gathers(prows_v, pgsem)
          pltpu.async_copy(
              prows_v,
              out_hbm.at[pl.ds(prb, _G), pl.ds(pcb, _EMBED)],
              pwsem)
      return carry

    lax.fori_loop(0, _NIT, it, 0)

    lu = u0 + _PER_W - 1
    lfeat_v, lidx_v, lrows_v, lfsem, lgsem, lwsem = sets[_NSET - 1]
    lfoff, lrb, lcb, _ = _unit_params(lu, k)
    drain_gathers(lrows_v, lgsem)
    pltpu.async_copy(
        lrows_v, out_hbm.at[pl.ds(lrb, _G), pl.ds(lcb, _EMBED)], lwsem)
    for s in range(_NSET):
      feat_v, idx_v, rows_v, fsem, gsem, wsem = sets[s]
      u = u0 + (_NIT - 1) * _NSET + s
      _, rb, cb, _ = _unit_params(u, k)
      pltpu.make_async_copy(
          rows_v, out_hbm.at[pl.ds(rb, _G), pl.ds(cb, _EMBED)], wsem).wait()

  return body(feat_t, table)


_TCB = 4


def _tc_transpose_first(x_ref, o_ref):
  for j in range(_TCB):
    y = x_ref[j].T
    o_ref[j, :, :_HB] = y[:_EMBED]
    o_ref[j, :, _HB:] = y[_EMBED:]


def _tc_transpose_next(acc_ref, x_ref, o_ref):
  del acc_ref
  _tc_transpose_first(x_ref, o_ref)


def _tc_transpose(g3, k, acc):
  out_spec = pl.BlockSpec(
      (_TCB, _EMBED, _NB), lambda i, _k=k: (_k * _CSLAB // _TCB + i, 0, 0))
  x_spec = pl.BlockSpec((_TCB, _HB, 128), lambda i: (i, 0, 0))
  out_shape = jax.ShapeDtypeStruct((_NSLAB, _EMBED, _NB), jnp.float32)
  if acc is None:
    return pl.pallas_call(
        _tc_transpose_first,
        grid=(_CSLAB // _TCB,),
        in_specs=[x_spec],
        out_specs=out_spec,
        out_shape=out_shape,
    )(g3)
  return pl.pallas_call(
      _tc_transpose_next,
      grid=(_CSLAB // _TCB,),
      in_specs=[pl.BlockSpec(memory_space=pl.ANY), x_spec],
      out_specs=out_spec,
      out_shape=out_shape,
      input_output_aliases={0: 0},
  )(acc, g3)


def kernel(feat, table):
  feat_t = feat.transpose(2, 1, 0).reshape(-1)
  acc = None
  for k in range(_K):
    g = _sc_gather_chunk(feat_t, table, k)
    g3 = g.reshape(_CSLAB, _HB, 128)
    acc = _tc_transpose(g3, k, acc)
  return acc.reshape(_NL, _MULT, _EMBED, _NB).transpose(3, 0, 1, 2)


# speedup vs baseline: 5.6395x; 1.0047x over previous
import functools

import jax
import jax.numpy as jnp
from jax import lax
from jax.experimental import pallas as pl
from jax.experimental.pallas import tpu as pltpu
from jax.experimental.pallas import tpu_sc as plsc

_NUM_CLASSES = 100000
_MULT = 3
_EMBED = 64

_NB = 4096
_NL = 200
_NSLAB = _NL * _MULT
_B = _NB * _NSLAB
_NC = 2
_NSC = 16
_NW = _NC * _NSC
_G = 512
_NT = _G // 128
_HB = _NB // 2
_K = 5
_CSLAB = _NSLAB // _K
_CUNITS = _CSLAB * (_NB // _G)
_PER_W = _CUNITS // _NW
_NSET = 3
_NIT = _PER_W // _NSET


def _unit_params(u, k):
  h = u >> 3
  l = (h * 43691) >> 17
  m = h - 3 * l
  bc = u & 7
  foff = (m * _NL + l) * _NB + bc * _G
  rb = (h - k * _CSLAB) * _HB + (bc & 3) * _G
  cb = (bc >> 2) * _EMBED
  bias = m * _NUM_CLASSES
  return foff, rb, cb, bias


def _sc_gather_chunk(feat_t, table, k):
  mesh = plsc.VectorSubcoreMesh(core_axis_name="c", subcore_axis_name="s")

  scratch = []
  for _ in range(_NSET):
    scratch += [
        pltpu.VMEM((_G,), jnp.int32),
        pltpu.VMEM((_NT, 128), jnp.int32),
        pltpu.VMEM((_G, _EMBED), jnp.float32),
        pltpu.SemaphoreType.DMA,
        pltpu.SemaphoreType.DMA,
        pltpu.SemaphoreType.DMA,
    ]

  @functools.partial(
      pl.kernel,
      mesh=mesh,
      out_type=jax.ShapeDtypeStruct((_CSLAB * _HB, 128), jnp.float32),
      scratch_types=scratch,
      compiler_params=pltpu.CompilerParams(use_tc_tiling_on_sc=False),
  )
  def body(feat_hbm, table_hbm, out_hbm, *bufs):
    sets = [bufs[i * 6:(i + 1) * 6] for i in range(_NSET)]
    wid = lax.axis_index("s") * _NC + lax.axis_index("c")
    u0 = k * _CUNITS + wid * _PER_W

    def prefetch_feat(u, feat_v, fsem):
      foff, _, _, _ = _unit_params(u, k)
      pltpu.async_copy(feat_hbm.at[pl.ds(foff, _G)], feat_v, fsem)

    def fire_gathers(idx_v, rows_v, gsem):
      for t in range(_NT):
        pltpu.async_copy(
            table_hbm.at[idx_v.at[t]],
            rows_v.at[pl.ds(t * 128, 128)],
            gsem,
        )

    def drain_gathers(rows_v, gsem):
      pltpu.make_async_copy(table_hbm.at[pl.ds(0, _G)], rows_v, gsem).wait()

    for s in range(_NSET):
      prefetch_feat(u0 + s, sets[s][0], sets[s][3])

    def it(i, carry):
      for s in range(_NSET):
        feat_v, idx_v, rows_v, fsem, gsem, wsem = sets[s]
        u = u0 + i * _NSET + s
        foff, rb, cb, bias = _unit_params(u, k)
        pltpu.make_async_copy(feat_hbm.at[pl.ds(foff, _G)], feat_v,
                              fsem).wait()
        for q in range(_G // 16):
          idx_v[q // 8, pl.ds((q % 8) * 16, 16)] = (
              feat_v[pl.ds(q * 16, 16)] + bias)

        @pl.when(i < _NIT - 1)
        def _():
          prefetch_feat(u + _NSET, feat_v, fsem)

        @pl.when(i > 0)
        def _():
          _, prb, pcb, _ = _unit_params(u - _NSET, k)
          pltpu.make_async_copy(
              rows_v,
              out_hbm.at[pl.ds(prb, _G), pl.ds(pcb, _EMBED)],
              wsem).wait()

        fire_gathers(idx_v, rows_v, gsem)

        ps = (s + _NSET - 1) % _NSET
        pfeat_v, pidx_v, prows_v, pfsem, pgsem, pwsem = sets[ps]

        @pl.when((i > 0) | (s > 0))
        def _():
          pu = u - 1
          _, prb, pcb, _ = _unit_params(pu, k)
          drain_gathers(prows_v, pgsem)
          pltpu.async_copy(
              prows_v,
              out_hbm.at[pl.ds(prb, _G), pl.ds(pcb, _EMBED)],
              pwsem)
      return carry

    lax.fori_loop(0, _NIT, it, 0)

    lu = u0 + _PER_W - 1
    lfeat_v, lidx_v, lrows_v, lfsem, lgsem, lwsem = sets[_NSET - 1]
    lfoff, lrb, lcb, _ = _unit_params(lu, k)
    drain_gathers(lrows_v, lgsem)
    pltpu.async_copy(
        lrows_v, out_hbm.at[pl.ds(lrb, _G), pl.ds(lcb, _EMBED)], lwsem)
    for s in range(_NSET):
      feat_v, idx_v, rows_v, fsem, gsem, wsem = sets[s]
      u = u0 + (_NIT - 1) * _NSET + s
      _, rb, cb, _ = _unit_params(u, k)
      pltpu.make_async_copy(
          rows_v, out_hbm.at[pl.ds(rb, _G), pl.ds(cb, _EMBED)], wsem).wait()

  return body(feat_t, table)


_TCB = 8


def _tc_transpose_first(x_ref, o_ref):
  for j in range(_TCB):
    y = x_ref[j].T
    o_ref[j, :, :_HB] = y[:_EMBED]
    o_ref[j, :, _HB:] = y[_EMBED:]


def _tc_transpose_next(acc_ref, x_ref, o_ref):
  del acc_ref
  _tc_transpose_first(x_ref, o_ref)


def _tc_transpose(g3, k, acc):
  out_spec = pl.BlockSpec(
      (_TCB, _EMBED, _NB), lambda i, _k=k: (_k * _CSLAB // _TCB + i, 0, 0))
  x_spec = pl.BlockSpec((_TCB, _HB, 128), lambda i: (i, 0, 0))
  out_shape = jax.ShapeDtypeStruct((_NSLAB, _EMBED, _NB), jnp.float32)
  if acc is None:
    return pl.pallas_call(
        _tc_transpose_first,
        grid=(_CSLAB // _TCB,),
        in_specs=[x_spec],
        out_specs=out_spec,
        out_shape=out_shape,
    )(g3)
  return pl.pallas_call(
      _tc_transpose_next,
      grid=(_CSLAB // _TCB,),
      in_specs=[pl.BlockSpec(memory_space=pl.ANY), x_spec],
      out_specs=out_spec,
      out_shape=out_shape,
      input_output_aliases={0: 0},
  )(acc, g3)


def kernel(feat, table):
  feat_t = feat.transpose(2, 1, 0).reshape(-1)
  acc = None
  for k in range(_K):
    g = _sc_gather_chunk(feat_t, table, k)
    g3 = g.reshape(_CSLAB, _HB, 128)
    acc = _tc_transpose(g3, k, acc)
  return acc.reshape(_NL, _MULT, _EMBED, _NB).transpose(3, 0, 1, 2)
